# baseline (speedup 1.0000x reference)
import jax
import jax.numpy as jnp
from jax.experimental import pallas as pl
from jax.experimental.pallas import tpu as pltpu

N = 2048
DIN = 256
DOUT = 256
R = 4
H = 4
DH = DOUT // H
BM = 512
NB = N // BM
BW = 384


def _rgat_kernel(feat_ref, adj_ref, w_ref, sel_ref, bias_ref,
                 out_ref, b_scr, es_scr):
    i = pl.program_id(0)
    r = pl.program_id(1)

    row128 = jax.lax.broadcasted_iota(jnp.int32, (128, DOUT), 0)
    col256 = jax.lax.broadcasted_iota(jnp.int32, (128, DOUT), 1)
    g2 = (row128 == col256 // DH).astype(jnp.float32)

    @pl.when(i == 0)
    def _prep():
        feat = feat_ref[...]
        w = w_ref[0]
        t = jnp.dot(feat, w.T, preferred_element_type=jnp.float32)
        sc = jnp.dot(t, sel_ref[0], preferred_element_type=jnp.float32)
        c128 = jax.lax.broadcasted_iota(jnp.int32, (N, 128), 1)
        lo = jnp.where(c128 < DH, sc[:, H:H + 1], sc[:, H + 1:H + 2])
        hi = jnp.where(c128 < DH, sc[:, H + 2:H + 3], sc[:, H + 3:H + 4])
        sdb = jnp.concatenate([lo, hi], axis=1)
        b_scr[r, :, 0:DOUT] = (jnp.exp(sdb) * t).astype(jnp.bfloat16)
        vcols = jnp.where(c128 == 2 * H, 1.0, 0.0)
        for h in range(H):
            vcols += jnp.where(c128 == h,
                               jnp.exp(sc[:, H + h:H + h + 1]), 0.0)
        b_scr[r, :, DOUT:BW] = vcols.astype(jnp.bfloat16)
        es_scr[r, :, :] = jnp.exp(sc)

    a_blk = adj_ref[0].astype(jnp.bfloat16)
    p = jnp.dot(a_blk, b_scr[r], preferred_element_type=jnp.float32)
    m = p[:, 0:DOUT]
    pc = p[:, DOUT:BW]
    es_c = es_scr[r, pl.ds(i * BM, BM), :]
    deg = p[:, DOUT + 2 * H:DOUT + 2 * H + 1]
    lane = jax.lax.broadcasted_iota(jnp.int32, (BM, 128), 1)
    z_c = es_c * pc + (jnp.float32(N) - deg)
    w_c = jnp.where(lane < H, es_c / z_c * jnp.float32(1.0 / R), 0.0)
    wb = jnp.dot(w_c, g2, preferred_element_type=jnp.float32)
    contrib = m * wb

    @pl.when(r == 0)
    def _init():
        out_ref[...] = contrib + bias_ref[...]

    @pl.when(r > 0)
    def _acc():
        out_ref[...] += contrib


def kernel(features, adjacency_matrices, W_rel, a_rel, bias):
    rows = jnp.arange(DOUT)
    cols = jnp.arange(128)
    smask = (cols[None, :] == rows[:, None] // DH).astype(jnp.float32)
    dmask = (cols[None, :] == rows[:, None] // DH + H).astype(jnp.float32)
    adst_t = jnp.tile(a_rel[:, DH:], (1, H))
    asrc_t = jnp.tile(a_rel[:, :DH], (1, H))
    sel = asrc_t[:, :, None] * smask[None] + adst_t[:, :, None] * dmask[None]
    bias2d = bias.reshape(1, DOUT)

    grid = (NB, R)
    out = pl.pallas_call(
        _rgat_kernel,
        grid=grid,
        in_specs=[
            pl.BlockSpec((N, DIN), lambda i, r: (0, 0)),
            pl.BlockSpec((1, BM, N), lambda i, r: (r, i, 0)),
            pl.BlockSpec((1, DOUT, DIN), lambda i, r: (r, 0, 0)),
            pl.BlockSpec((1, DOUT, 128), lambda i, r: (r, 0, 0)),
            pl.BlockSpec((1, DOUT), lambda i, r: (0, 0)),
        ],
        out_specs=pl.BlockSpec((BM, DOUT), lambda i, r: (i, 0)),
        out_shape=jax.ShapeDtypeStruct((N, DOUT), jnp.float32),
        scratch_shapes=[
            pltpu.VMEM((R, N, BW), jnp.bfloat16),
            pltpu.VMEM((R, N, 128), jnp.float32),
        ],
        compiler_params=pltpu.CompilerParams(
            dimension_semantics=("arbitrary", "arbitrary"),
        ),
    )(features, adjacency_matrices, W_rel, sel, bias2d)
    return out


# speedup vs baseline: 1.1254x; 1.1254x over previous
import jax
import jax.numpy as jnp
from jax.experimental import pallas as pl
from jax.experimental.pallas import tpu as pltpu

N = 2048
DIN = 256
DOUT = 256
R = 4
H = 4
DH = DOUT // H
BM = 512
NB = N // BM
BW = 384


def _rgat_kernel(feat_ref, adj_ref, w_ref, sel_ref, bias_ref,
                 out_ref, b_scr, es_scr):
    i = pl.program_id(0)
    r = pl.program_id(1)

    row128 = jax.lax.broadcasted_iota(jnp.int32, (128, DOUT), 0)
    col256 = jax.lax.broadcasted_iota(jnp.int32, (128, DOUT), 1)
    g2 = (row128 == col256 // DH).astype(jnp.float32)

    @pl.when(i == 0)
    def _prep():
        feat = feat_ref[...]
        w = w_ref[0]
        t = jnp.dot(feat, w.T, preferred_element_type=jnp.float32)
        sc = jnp.dot(t, sel_ref[0], preferred_element_type=jnp.float32)
        esc = jnp.exp(sc)
        es_scr[r, :, :] = esc
        rowg = jax.lax.broadcasted_iota(jnp.int32, (128, DOUT), 0)
        colg = jax.lax.broadcasted_iota(jnp.int32, (128, DOUT), 1)
        gd2 = (rowg == colg // DH + H).astype(jnp.float32)
        vb = jnp.dot(esc, gd2, preferred_element_type=jnp.float32)
        b_scr[r, :, 0:DOUT] = (vb * t).astype(jnp.bfloat16)
        rs = jax.lax.broadcasted_iota(jnp.int32, (128, 128), 0)
        cs = jax.lax.broadcasted_iota(jnp.int32, (128, 128), 1)
        gsh = ((rs == cs + H) & (cs < H)).astype(jnp.float32)
        c128 = jax.lax.broadcasted_iota(jnp.int32, (N, 128), 1)
        vcols = jnp.dot(esc, gsh, preferred_element_type=jnp.float32) \
            + jnp.where(c128 == 2 * H, 1.0, 0.0)
        b_scr[r, :, DOUT:BW] = vcols.astype(jnp.bfloat16)

    a_blk = adj_ref[0].astype(jnp.bfloat16)
    p = jnp.dot(a_blk, b_scr[r], preferred_element_type=jnp.float32)
    m = p[:, 0:DOUT]
    pc = p[:, DOUT:BW]
    es_c = es_scr[r, pl.ds(i * BM, BM), :]
    deg = p[:, DOUT + 2 * H:DOUT + 2 * H + 1]
    lane = jax.lax.broadcasted_iota(jnp.int32, (BM, 128), 1)
    z_c = es_c * pc + (jnp.float32(N) - deg)
    w_c = jnp.where(lane < H, es_c / z_c * jnp.float32(1.0 / R), 0.0)
    wb = jnp.dot(w_c, g2, preferred_element_type=jnp.float32)
    contrib = m * wb

    @pl.when(r == 0)
    def _init():
        out_ref[...] = contrib + bias_ref[...]

    @pl.when(r > 0)
    def _acc():
        out_ref[...] += contrib


def kernel(features, adjacency_matrices, W_rel, a_rel, bias):
    rows = jnp.arange(DOUT)
    cols = jnp.arange(128)
    smask = (cols[None, :] == rows[:, None] // DH).astype(jnp.float32)
    dmask = (cols[None, :] == rows[:, None] // DH + H).astype(jnp.float32)
    adst_t = jnp.tile(a_rel[:, DH:], (1, H))
    asrc_t = jnp.tile(a_rel[:, :DH], (1, H))
    sel = asrc_t[:, :, None] * smask[None] + adst_t[:, :, None] * dmask[None]
    bias2d = bias.reshape(1, DOUT)

    grid = (NB, R)
    out = pl.pallas_call(
        _rgat_kernel,
        grid=grid,
        in_specs=[
            pl.BlockSpec((N, DIN), lambda i, r: (0, 0)),
            pl.BlockSpec((1, BM, N), lambda i, r: (r, i, 0)),
            pl.BlockSpec((1, DOUT, DIN), lambda i, r: (r, 0, 0)),
            pl.BlockSpec((1, DOUT, 128), lambda i, r: (r, 0, 0)),
            pl.BlockSpec((1, DOUT), lambda i, r: (0, 0)),
        ],
        out_specs=pl.BlockSpec((BM, DOUT), lambda i, r: (i, 0)),
        out_shape=jax.ShapeDtypeStruct((N, DOUT), jnp.float32),
        scratch_shapes=[
            pltpu.VMEM((R, N, BW), jnp.bfloat16),
            pltpu.VMEM((R, N, 128), jnp.float32),
        ],
        compiler_params=pltpu.CompilerParams(
            dimension_semantics=("arbitrary", "arbitrary"),
        ),
    )(features, adjacency_matrices, W_rel, sel, bias2d)
    return out
